# baseline (speedup 1.0000x reference)
import functools

import jax
import jax.numpy as jnp
from jax.experimental import pallas as pl
from jax.experimental.pallas import tpu as pltpu

N = 1024
D = 128
BK = 2048
CK = 512
NC = BK // CK


def _fold_kernel(emb_ref, w_ref, b_ref, tab_ref, out_ref, *scratch,
                 nsteps):
    best_refs = scratch[:NC]
    blk_refs = scratch[NC:]
    j = pl.program_id(0)

    y = (emb_ref[...] - b_ref[...]) / (w_ref[...] + 1e-6)

    ones_row = jnp.ones((1, D), jnp.float32)
    contract = (((1,), (1,)), ((), ()))

    def dot(a, b):
        return jax.lax.dot_general(a, b, contract,
                                   preferred_element_type=jnp.float32)

    for c in range(NC):
        tb = tab_ref[c * CK:(c + 1) * CK, :]
        tsq = tb * tb
        h1 = tsq.astype(jnp.bfloat16).astype(jnp.float32)
        r1 = tsq - h1
        h2 = r1.astype(jnp.bfloat16).astype(jnp.float32)
        h3 = r1 - h2
        t2h = 0.5 * ((dot(ones_row, h1) + dot(ones_row, h2))
                     + dot(ones_row, h3))
        mm = dot(y, tb)
        s = t2h - mm

        prev = jnp.where(j == 0, jnp.float32(jnp.inf), best_refs[c][...])
        upd = s < prev
        best_refs[c][...] = jnp.minimum(s, prev)
        blk_refs[c][...] = jnp.where(upd, j, blk_refs[c][...])

    @pl.when(j == nsteps - 1)
    def _done():
        big = jnp.int32(2147483647)
        lane = jax.lax.broadcasted_iota(jnp.int32, (1, CK), 1)
        rowmin = jnp.min(best_refs[0][...], axis=1, keepdims=True)
        for c in range(1, NC):
            rowmin = jnp.minimum(
                rowmin, jnp.min(best_refs[c][...], axis=1, keepdims=True))
        idx = jnp.full((N, 1), big, jnp.int32)
        for c in range(NC):
            gcol = blk_refs[c][...] * BK + (c * CK) + lane
            cand = jnp.where(best_refs[c][...] == rowmin, gcol, big)
            idx = jnp.minimum(idx, jnp.min(cand, axis=1, keepdims=True))
        out_ref[...] = idx


@jax.jit
def kernel(embeddings, ln_weight, ln_bias, table):
    vocab = table.shape[0]
    nsteps = pl.cdiv(vocab, BK)
    padded = nsteps * BK
    if padded != vocab:
        table = jnp.pad(table, ((0, padded - vocab), (0, 0)), mode="edge")

    scratch = ([pltpu.VMEM((N, CK), jnp.float32) for _ in range(NC)]
               + [pltpu.VMEM((N, CK), jnp.int32) for _ in range(NC)])
    out = pl.pallas_call(
        functools.partial(_fold_kernel, nsteps=nsteps),
        grid=(nsteps,),
        in_specs=[
            pl.BlockSpec((N, D), lambda j: (0, 0)),
            pl.BlockSpec((1, D), lambda j: (0, 0)),
            pl.BlockSpec((1, D), lambda j: (0, 0)),
            pl.BlockSpec((BK, D), lambda j: (j, 0)),
        ],
        out_specs=pl.BlockSpec((N, 1), lambda j: (0, 0)),
        out_shape=jax.ShapeDtypeStruct((N, 1), jnp.int32),
        scratch_shapes=scratch,
    )(embeddings, ln_weight[None, :], ln_bias[None, :], table)
    return out[:, 0]


# speedup vs baseline: 1.1191x; 1.1191x over previous
import functools

import jax
import jax.numpy as jnp
from jax.experimental import pallas as pl
from jax.experimental.pallas import tpu as pltpu

N = 1024
D = 128
BK = 2048


def _fold_kernel(emb_ref, w_ref, b_ref, tab_ref, out_ref, best_ref, blk_ref,
                 *, nsteps, blk):
    j = pl.program_id(0)

    tb = tab_ref[...]
    ones_row = jnp.ones((1, D), jnp.float32)
    contract = (((1,), (1,)), ((), ()))
    tsq = tb * tb
    h1 = tsq.astype(jnp.bfloat16).astype(jnp.float32)
    r1 = tsq - h1
    h2 = r1.astype(jnp.bfloat16).astype(jnp.float32)
    h3 = r1 - h2
    dot = lambda a: jax.lax.dot_general(ones_row, a, contract,
                                        preferred_element_type=jnp.float32)
    t2h = 0.5 * ((dot(h1) + dot(h2)) + dot(h3))

    y = (emb_ref[...] - b_ref[...]) / (w_ref[...] + 1e-6)

    mm = jax.lax.dot_general(y, tb, contract,
                             preferred_element_type=jnp.float32)
    s = t2h - mm

    prev = jnp.where(j == 0, jnp.float32(jnp.inf), best_ref[...])
    upd = s < prev
    best_ref[...] = jnp.minimum(s, prev)
    blk_ref[...] = jnp.where(upd, j.astype(jnp.int16), blk_ref[...])

    @pl.when(j == nsteps - 1)
    def _done():
        m = best_ref[...]
        rowmin = jnp.min(m, axis=1, keepdims=True)
        lane = jax.lax.broadcasted_iota(jnp.int32, (1, blk), 1)
        gcol = blk_ref[...].astype(jnp.int32) * blk + lane
        big = jnp.int32(2147483647)
        cand = jnp.where(m == rowmin, gcol, big)
        out_ref[...] = jnp.min(cand, axis=1, keepdims=True)


@jax.jit
def kernel(embeddings, ln_weight, ln_bias, table):
    vocab = table.shape[0]
    nsteps = pl.cdiv(vocab, BK)
    padded = nsteps * BK
    if padded != vocab:
        table = jnp.pad(table, ((0, padded - vocab), (0, 0)), mode="edge")

    out = pl.pallas_call(
        functools.partial(_fold_kernel, nsteps=nsteps, blk=BK),
        grid=(nsteps,),
        in_specs=[
            pl.BlockSpec((N, D), lambda j: (0, 0)),
            pl.BlockSpec((1, D), lambda j: (0, 0)),
            pl.BlockSpec((1, D), lambda j: (0, 0)),
            pl.BlockSpec((BK, D), lambda j: (j, 0)),
        ],
        out_specs=pl.BlockSpec((N, 1), lambda j: (0, 0)),
        out_shape=jax.ShapeDtypeStruct((N, 1), jnp.int32),
        scratch_shapes=[
            pltpu.VMEM((N, BK), jnp.float32),
            pltpu.VMEM((N, BK), jnp.int16),
        ],
    )(embeddings, ln_weight[None, :], ln_bias[None, :], table)
    return out[:, 0]
